# baseline (speedup 1.0000x reference)
import functools

import jax
import jax.numpy as jnp
from jax import lax
from jax.experimental import pallas as pl
from jax.experimental.pallas import tpu as pltpu
from jax.experimental.pallas import tpu_sc as plsc

N = 10000
E = 320000
NC, NS = 2, 16
NW = NC * NS
CH = 128
NCH = 80
EPAD = NW * NCH * CH
ACC_N = 10240
RPS = ACC_N // NS
ACC_ROWS = ACC_N

NCHT = 160
SLOW = 0

ROW_BLK = 2000
GRID = N // ROW_BLK

_sc_mesh = plsc.VectorSubcoreMesh(core_axis_name="c", subcore_axis_name="s")
_sc_params = pltpu.CompilerParams(use_tc_tiling_on_sc=False)


def _make_agg(F, na, nb, NBUF, GDEPTH):
    assert na % NBUF == 0 and nb % NBUF == 0 and na + nb == NCHT

    @functools.partial(
        pl.kernel,
        out_type=jax.ShapeDtypeStruct((NC * ACC_N, F), jnp.float32),
        mesh=_sc_mesh,
        compiler_params=_sc_params,
        scratch_types=[
            pltpu.VMEM((max(na, nb), CH), jnp.int32),
            pltpu.VMEM((max(na, nb), CH), jnp.int32),
            pltpu.VMEM((NBUF, CH, F), jnp.float32),
            pltpu.VMEM_SHARED((ACC_ROWS, F), jnp.float32),
        ] + [pltpu.SemaphoreType.DMA] * (2 * NBUF),
    )
    def agg(hs_hbm, srcr_hbm, dstr_hbm, zeros_hbm, out_hbm,
            src_v, dst_v, rows_v, acc_sh, *sems):
        gsems, ssems = sems[:NBUF], sems[NBUF:]
        c = lax.axis_index("c")
        s = lax.axis_index("s")
        pltpu.sync_copy(zeros_hbm, acc_sh.at[pl.ds(s * RPS, RPS)])

        def run(nch, base):
            pltpu.sync_copy(srcr_hbm.at[pl.ds(base, nch)],
                            src_v.at[pl.ds(0, nch)])
            pltpu.sync_copy(dstr_hbm.at[pl.ds(base, nch)],
                            dst_v.at[pl.ds(0, nch)])
            plsc.subcore_barrier()
            for k in range(GDEPTH):
                pltpu.async_copy(hs_hbm.at[src_v.at[k]], rows_v.at[k],
                                 gsems[k])

            @pl.loop(0, nch, step=NBUF)
            def _(j0):
                for b in range(NBUF):
                    j = j0 + b
                    pltpu.make_async_copy(
                        hs_hbm.at[src_v.at[j]], rows_v.at[b], gsems[b]).wait()
                    pltpu.async_copy(
                        rows_v.at[b], acc_sh.at[dst_v.at[j]], ssems[b],
                        add=True)
                    bn = (b + GDEPTH) % NBUF
                    jn = j + GDEPTH

                    @pl.when(jn < nch)
                    def _():
                        @pl.when(jn >= NBUF)
                        def _():
                            pltpu.make_async_copy(
                                rows_v.at[bn], acc_sh.at[dst_v.at[j]],
                                ssems[bn]).wait()

                        pltpu.async_copy(
                            hs_hbm.at[src_v.at[jn]], rows_v.at[bn], gsems[bn])

            for b in range(NBUF):
                pltpu.make_async_copy(
                    rows_v.at[b], acc_sh.at[dst_v.at[b]], ssems[b]).wait()

        @pl.when(c == SLOW)
        def _():
            run(na, s * na)

        @pl.when(c != SLOW)
        def _():
            run(nb, NS * na + s * nb)

        plsc.subcore_barrier()
        pltpu.sync_copy(acc_sh.at[pl.ds(s * RPS, RPS)],
                        out_hbm.at[pl.ds(c * ACC_N + s * RPS, RPS)])

    return agg


_agg = {64: _make_agg(64, 24, 136, 4, 4),
        32: _make_agg(32, 40, 120, 8, 6),
        16: _make_agg(16, 48, 112, 8, 6)}


@functools.partial(
    pl.kernel,
    out_type=jax.ShapeDtypeStruct((NC * ACC_N, 16), jnp.float32),
    mesh=_sc_mesh,
    compiler_params=_sc_params,
    scratch_types=[
        pltpu.VMEM((96, CH), jnp.int32),
        pltpu.VMEM((CH, 16), jnp.float32),
        pltpu.VMEM_SHARED((ACC_ROWS, 16), jnp.float32),
        pltpu.SemaphoreType.DMA,
    ],
)
def _deg_kernel(dstr_hbm, zeros_hbm, ones_hbm, out_hbm, dst_v, ones_v, acc_sh,
                ssem):
    c = lax.axis_index("c")
    s = lax.axis_index("s")
    pltpu.sync_copy(zeros_hbm, acc_sh.at[pl.ds(s * RPS, RPS)])
    pltpu.sync_copy(ones_hbm, ones_v)

    def run(nch, base):
        pltpu.sync_copy(dstr_hbm.at[pl.ds(base, nch)], dst_v.at[pl.ds(0, nch)])
        plsc.subcore_barrier()

        @pl.loop(0, nch)
        def _(j):
            pltpu.async_copy(ones_v, acc_sh.at[dst_v.at[j]], ssem, add=True)

        @pl.loop(0, nch)
        def _(j):
            pltpu.make_async_copy(ones_v, acc_sh.at[dst_v.at[j]], ssem).wait()

    @pl.when(c == SLOW)
    def _():
        run(64, s * 64)

    @pl.when(c != SLOW)
    def _():
        run(96, NS * 64 + s * 96)

    plsc.subcore_barrier()
    pltpu.sync_copy(acc_sh.at[pl.ds(s * RPS, RPS)],
                    out_hbm.at[pl.ds(c * ACC_N + s * RPS, RPS)])


def _row_spec(width):
    return pl.BlockSpec((ROW_BLK, width), lambda i: (i, 0))


def _full_spec(shape):
    return pl.BlockSpec(shape, lambda i: (0, 0))


def _tc_call(body, in_specs, out_widths):
    out_shape = tuple(jax.ShapeDtypeStruct((N, w), jnp.float32) for w in out_widths)
    out_specs = tuple(_row_spec(w) for w in out_widths)
    if len(out_widths) == 1:
        out_shape, out_specs = out_shape[0], out_specs[0]
    return pl.pallas_call(
        body, grid=(GRID,), in_specs=in_specs,
        out_specs=out_specs, out_shape=out_shape)


def _t1_body(x_ref, w_ref, o_ref):
    o_ref[...] = jnp.dot(x_ref[...], w_ref[...],
                         preferred_element_type=jnp.float32)


def _f1_body(d0_ref, d1_ref, t1_ref, o_dinv, o_f1):
    deg = d0_ref[...] + d1_ref[...] + 1.0
    dinv = lax.rsqrt(deg)
    o_dinv[...] = dinv
    o_f1[...] = dinv * t1_ref[...]


def _post1_body(a0, a1, t1, dinv, b1, o_h1, o_f2):
    dv = dinv[...]
    g = dv * (a0[...] + a1[...]) + dv * dv * t1[...] + b1[...]
    h1 = jnp.maximum(g, 0.0)
    o_h1[...] = h1
    o_f2[...] = dv * h1


def _post2_body(a0, a1, h1, dinv, w2, b2, w3, o_t3, o_f3):
    dv = dinv[...]
    g2 = dv * (a0[...] + a1[...]) + dv * dv * h1[...]
    h2 = jnp.maximum(
        jnp.dot(g2, w2[...], preferred_element_type=jnp.float32) + b2[...], 0.0)
    t3 = jnp.dot(h2, w3[...], preferred_element_type=jnp.float32)
    o_t3[...] = t3
    o_f3[...] = dv * t3


def _post3_body(a0, a1, t3, dinv, b3, w4, o_t4, o_f4):
    dv = dinv[...]
    h3 = jnp.maximum(dv * (a0[...] + a1[...]) + dv * dv * t3[...] + b3[...], 0.0)
    t4 = jnp.dot(h3, w4[...], preferred_element_type=jnp.float32)
    o_t4[...] = t4
    o_f4[...] = dv * t4


def _post4_body(a0, a1, t4, dinv, b4, o_h4, o_f5):
    dv = dinv[...]
    h4 = jnp.maximum(dv * (a0[...] + a1[...]) + dv * dv * t4[...] + b4[...], 0.0)
    o_h4[...] = h4
    o_f5[...] = dv * h4


def _post5_body(a0, a1, h4, dinv, w5, b5, o_ref):
    dv = dinv[...]
    g5 = dv * (a0[...] + a1[...]) + dv * dv * h4[...]
    o_ref[...] = jnp.dot(g5, w5[...], preferred_element_type=jnp.float32) + b5[...]


def kernel(x, edge_index, W1, b1, W2, b2, W3, b3, W4, b4, W5, b5):
    src = edge_index[0]
    dst = edge_index[1]
    pad = EPAD - E
    srcr = jnp.concatenate(
        [src, jnp.zeros((pad,), src.dtype)]).reshape(NW * NCH, CH)
    dstr = jnp.concatenate(
        [dst, jnp.full((pad,), N, dst.dtype)]).reshape(NW * NCH, CH)

    zeros64 = jnp.zeros((RPS, 64), jnp.float32)
    zeros32 = jnp.zeros((RPS, 32), jnp.float32)
    zeros16 = jnp.zeros((RPS, 16), jnp.float32)
    ones16 = jnp.ones((CH, 16), jnp.float32)

    degp = _deg_kernel(dstr, zeros16, ones16)
    t1 = _tc_call(_t1_body, [_row_spec(128), _full_spec((128, 64))], (64,))(x, W1)

    d0 = degp[:N, :1]
    d1 = degp[ACC_N:ACC_N + N, :1]
    dinv, f1 = _tc_call(
        _f1_body, [_row_spec(1), _row_spec(1), _row_spec(64)], (1, 64),
    )(d0, d1, t1)

    a1 = _agg[64](f1, srcr, dstr, zeros64)
    h1, f2 = _tc_call(
        _post1_body,
        [_row_spec(64), _row_spec(64), _row_spec(64), _row_spec(1),
         _full_spec((1, 64))],
        (64, 64),
    )(a1[:N], a1[ACC_N:ACC_N + N], t1, dinv, b1.reshape(1, -1))

    a2 = _agg[64](f2, srcr, dstr, zeros64)
    t3, f3 = _tc_call(
        _post2_body,
        [_row_spec(64), _row_spec(64), _row_spec(64), _row_spec(1),
         _full_spec((64, 256)), _full_spec((1, 256)), _full_spec((256, 32))],
        (32, 32),
    )(a2[:N], a2[ACC_N:ACC_N + N], h1, dinv, W2, b2.reshape(1, -1), W3)

    a3 = _agg[32](f3, srcr, dstr, zeros32)
    t4, f4 = _tc_call(
        _post3_body,
        [_row_spec(32), _row_spec(32), _row_spec(32), _row_spec(1),
         _full_spec((1, 32)), _full_spec((32, 16))],
        (16, 16),
    )(a3[:N], a3[ACC_N:ACC_N + N], t3, dinv, b3.reshape(1, -1), W4)

    a4 = _agg[16](f4, srcr, dstr, zeros16)
    h4, f5 = _tc_call(
        _post4_body,
        [_row_spec(16), _row_spec(16), _row_spec(16), _row_spec(1),
         _full_spec((1, 16))],
        (16, 16),
    )(a4[:N], a4[ACC_N:ACC_N + N], t4, dinv, b4.reshape(1, -1))

    a5 = _agg[16](f5, srcr, dstr, zeros16)
    out = _tc_call(
        _post5_body,
        [_row_spec(16), _row_spec(16), _row_spec(16), _row_spec(1),
         _full_spec((16, 40)), _full_spec((1, 40))],
        (40,),
    )(a5[:N], a5[ACC_N:ACC_N + N], h4, dinv, W5, b5.reshape(1, -1))
    return out


# speedup vs baseline: 1.0796x; 1.0796x over previous
import functools

import jax
import jax.numpy as jnp
from jax import lax
from jax.experimental import pallas as pl
from jax.experimental.pallas import tpu as pltpu
from jax.experimental.pallas import tpu_sc as plsc

N = 10000
E = 320000
NC, NS = 2, 16
NW = NC * NS
CH = 128
NCH = 80
EPAD = NW * NCH * CH
ACC_N = 10240
RPS = ACC_N // NS
ACC_ROWS = ACC_N

NCHT = 160
SLOW = 0

ROW_BLK = 2000
GRID = N // ROW_BLK

_sc_mesh = plsc.VectorSubcoreMesh(core_axis_name="c", subcore_axis_name="s")
_sc_params = pltpu.CompilerParams(use_tc_tiling_on_sc=False)


def _make_agg(F, na, nb, NBUF, GDEPTH):
    assert na % NBUF == 0 and nb % NBUF == 0 and na + nb == NCHT

    @functools.partial(
        pl.kernel,
        out_type=jax.ShapeDtypeStruct((NC * ACC_N, F), jnp.float32),
        mesh=_sc_mesh,
        compiler_params=_sc_params,
        scratch_types=[
            pltpu.VMEM((max(na, nb), CH), jnp.int32),
            pltpu.VMEM((max(na, nb), CH), jnp.int32),
            pltpu.VMEM((NBUF, CH, F), jnp.float32),
            pltpu.VMEM_SHARED((ACC_ROWS, F), jnp.float32),
        ] + [pltpu.SemaphoreType.DMA] * (2 * NBUF),
    )
    def agg(hs_hbm, srcr_hbm, dstr_hbm, zeros_hbm, out_hbm,
            src_v, dst_v, rows_v, acc_sh, *sems):
        gsems, ssems = sems[:NBUF], sems[NBUF:]
        c = lax.axis_index("c")
        s = lax.axis_index("s")
        pltpu.sync_copy(zeros_hbm, acc_sh.at[pl.ds(s * RPS, RPS)])

        def run(nch, base):
            pltpu.sync_copy(srcr_hbm.at[pl.ds(base, nch)],
                            src_v.at[pl.ds(0, nch)])
            pltpu.sync_copy(dstr_hbm.at[pl.ds(base, nch)],
                            dst_v.at[pl.ds(0, nch)])
            plsc.subcore_barrier()
            for k in range(GDEPTH):
                pltpu.async_copy(hs_hbm.at[src_v.at[k]], rows_v.at[k],
                                 gsems[k])

            @pl.loop(0, nch, step=NBUF)
            def _(j0):
                for b in range(NBUF):
                    j = j0 + b
                    pltpu.make_async_copy(
                        hs_hbm.at[src_v.at[j]], rows_v.at[b], gsems[b]).wait()
                    pltpu.async_copy(
                        rows_v.at[b], acc_sh.at[dst_v.at[j]], ssems[b],
                        add=True)
                    bn = (b + GDEPTH) % NBUF
                    jn = j + GDEPTH

                    @pl.when(jn < nch)
                    def _():
                        @pl.when(jn >= NBUF)
                        def _():
                            pltpu.make_async_copy(
                                rows_v.at[bn], acc_sh.at[dst_v.at[j]],
                                ssems[bn]).wait()

                        pltpu.async_copy(
                            hs_hbm.at[src_v.at[jn]], rows_v.at[bn], gsems[bn])

            for b in range(NBUF):
                pltpu.make_async_copy(
                    rows_v.at[b], acc_sh.at[dst_v.at[b]], ssems[b]).wait()

        @pl.when(c == SLOW)
        def _():
            run(na, s * na)

        @pl.when(c != SLOW)
        def _():
            run(nb, NS * na + s * nb)

        plsc.subcore_barrier()
        pltpu.sync_copy(acc_sh.at[pl.ds(s * RPS, RPS)],
                        out_hbm.at[pl.ds(c * ACC_N + s * RPS, RPS)])

    return agg


_agg = {64: _make_agg(64, 24, 136, 4, 4),
        32: _make_agg(32, 40, 120, 8, 6),
        16: _make_agg(16, 48, 112, 8, 6)}


def _row_spec(width):
    return pl.BlockSpec((ROW_BLK, width), lambda i: (i, 0))


def _full_spec(shape):
    return pl.BlockSpec(shape, lambda i: (0, 0))


def _tc_call(body, in_specs, out_widths):
    out_shape = tuple(jax.ShapeDtypeStruct((N, w), jnp.float32) for w in out_widths)
    out_specs = tuple(_row_spec(w) for w in out_widths)
    if len(out_widths) == 1:
        out_shape, out_specs = out_shape[0], out_specs[0]
    return pl.pallas_call(
        body, grid=(GRID,), in_specs=in_specs,
        out_specs=out_specs, out_shape=out_shape)


def _t1_body(x_ref, w_ref, o_ref):
    o_ref[...] = jnp.dot(x_ref[...], w_ref[...],
                         preferred_element_type=jnp.float32)


def _f1_body(d0_ref, d1_ref, t1_ref, o_dinv, o_f1):
    deg = d0_ref[...] + d1_ref[...] + 1.0
    dinv = lax.rsqrt(deg)
    o_dinv[...] = dinv
    o_f1[...] = dinv * t1_ref[...]


def _post1_body(a0, a1, t1, dinv, b1, o_h1, o_f2):
    dv = dinv[...]
    g = dv * (a0[...] + a1[...]) + dv * dv * t1[...] + b1[...]
    h1 = jnp.maximum(g, 0.0)
    o_h1[...] = h1
    o_f2[...] = dv * h1


def _post2_body(a0, a1, h1, dinv, w2, b2, w3, o_t3, o_f3):
    dv = dinv[...]
    g2 = dv * (a0[...] + a1[...]) + dv * dv * h1[...]
    h2 = jnp.maximum(
        jnp.dot(g2, w2[...], preferred_element_type=jnp.float32) + b2[...], 0.0)
    t3 = jnp.dot(h2, w3[...], preferred_element_type=jnp.float32)
    o_t3[...] = t3
    o_f3[...] = dv * t3


def _post3_body(a0, a1, t3, dinv, b3, w4, o_t4, o_f4):
    dv = dinv[...]
    h3 = jnp.maximum(dv * (a0[...] + a1[...]) + dv * dv * t3[...] + b3[...], 0.0)
    t4 = jnp.dot(h3, w4[...], preferred_element_type=jnp.float32)
    o_t4[...] = t4
    o_f4[...] = dv * t4


def _post4_body(a0, a1, t4, dinv, b4, o_h4, o_f5):
    dv = dinv[...]
    h4 = jnp.maximum(dv * (a0[...] + a1[...]) + dv * dv * t4[...] + b4[...], 0.0)
    o_h4[...] = h4
    o_f5[...] = dv * h4


def _post5_body(a0, a1, h4, dinv, w5, b5, o_ref):
    dv = dinv[...]
    g5 = dv * (a0[...] + a1[...]) + dv * dv * h4[...]
    o_ref[...] = jnp.dot(g5, w5[...], preferred_element_type=jnp.float32) + b5[...]


def kernel(x, edge_index, W1, b1, W2, b2, W3, b3, W4, b4, W5, b5):
    src = edge_index[0]
    dst = edge_index[1]
    pad = EPAD - E
    srcr = jnp.concatenate(
        [src, jnp.zeros((pad,), src.dtype)]).reshape(NW * NCH, CH)
    dstr = jnp.concatenate(
        [dst, jnp.full((pad,), N, dst.dtype)]).reshape(NW * NCH, CH)

    zeros64 = jnp.zeros((RPS, 64), jnp.float32)
    zeros32 = jnp.zeros((RPS, 32), jnp.float32)
    zeros16 = jnp.zeros((RPS, 16), jnp.float32)
    ones16 = jnp.ones((N, 16), jnp.float32)

    degp = _agg[16](ones16, srcr, dstr, zeros16)
    t1 = _tc_call(_t1_body, [_row_spec(128), _full_spec((128, 64))], (64,))(x, W1)

    d0 = degp[:N, :1]
    d1 = degp[ACC_N:ACC_N + N, :1]
    dinv, f1 = _tc_call(
        _f1_body, [_row_spec(1), _row_spec(1), _row_spec(64)], (1, 64),
    )(d0, d1, t1)

    a1 = _agg[64](f1, srcr, dstr, zeros64)
    h1, f2 = _tc_call(
        _post1_body,
        [_row_spec(64), _row_spec(64), _row_spec(64), _row_spec(1),
         _full_spec((1, 64))],
        (64, 64),
    )(a1[:N], a1[ACC_N:ACC_N + N], t1, dinv, b1.reshape(1, -1))

    a2 = _agg[64](f2, srcr, dstr, zeros64)
    t3, f3 = _tc_call(
        _post2_body,
        [_row_spec(64), _row_spec(64), _row_spec(64), _row_spec(1),
         _full_spec((64, 256)), _full_spec((1, 256)), _full_spec((256, 32))],
        (32, 32),
    )(a2[:N], a2[ACC_N:ACC_N + N], h1, dinv, W2, b2.reshape(1, -1), W3)

    a3 = _agg[32](f3, srcr, dstr, zeros32)
    t4, f4 = _tc_call(
        _post3_body,
        [_row_spec(32), _row_spec(32), _row_spec(32), _row_spec(1),
         _full_spec((1, 32)), _full_spec((32, 16))],
        (16, 16),
    )(a3[:N], a3[ACC_N:ACC_N + N], t3, dinv, b3.reshape(1, -1), W4)

    a4 = _agg[16](f4, srcr, dstr, zeros16)
    h4, f5 = _tc_call(
        _post4_body,
        [_row_spec(16), _row_spec(16), _row_spec(16), _row_spec(1),
         _full_spec((1, 16))],
        (16, 16),
    )(a4[:N], a4[ACC_N:ACC_N + N], t4, dinv, b4.reshape(1, -1))

    a5 = _agg[16](f5, srcr, dstr, zeros16)
    out = _tc_call(
        _post5_body,
        [_row_spec(16), _row_spec(16), _row_spec(16), _row_spec(1),
         _full_spec((16, 40)), _full_spec((1, 40))],
        (40,),
    )(a5[:N], a5[ACC_N:ACC_N + N], h4, dinv, W5, b5.reshape(1, -1))
    return out


# speedup vs baseline: 1.1262x; 1.0431x over previous
import functools

import jax
import jax.numpy as jnp
from jax import lax
from jax.experimental import pallas as pl
from jax.experimental.pallas import tpu as pltpu
from jax.experimental.pallas import tpu_sc as plsc

N = 10000
E = 320000
NC, NS = 2, 16
NW = NC * NS
CH = 128
NCH = 80
EPAD = NW * NCH * CH
ACC_N = 10240
RPS = ACC_N // NS
ACC_ROWS = ACC_N

NCHT = 160
SLOW = 0

ROW_BLK = 2000
GRID = N // ROW_BLK

_sc_mesh = plsc.VectorSubcoreMesh(core_axis_name="c", subcore_axis_name="s")
_sc_params = pltpu.CompilerParams(use_tc_tiling_on_sc=False)


def _make_agg(F, na, nb, NBUF, GDEPTH):
    assert na % NBUF == 0 and nb % NBUF == 0 and na + nb == NCHT

    @functools.partial(
        pl.kernel,
        out_type=jax.ShapeDtypeStruct((NC * ACC_N, F), jnp.float32),
        mesh=_sc_mesh,
        compiler_params=_sc_params,
        scratch_types=[
            pltpu.VMEM((max(na, nb), CH), jnp.int32),
            pltpu.VMEM((max(na, nb), CH), jnp.int32),
            pltpu.VMEM((NBUF, CH, F), jnp.float32),
            pltpu.VMEM_SHARED((ACC_ROWS, F), jnp.float32),
        ] + [pltpu.SemaphoreType.DMA] * (2 * NBUF),
    )
    def agg(hs_hbm, srcr_hbm, dstr_hbm, zeros_hbm, out_hbm,
            src_v, dst_v, rows_v, acc_sh, *sems):
        gsems, ssems = sems[:NBUF], sems[NBUF:]
        c = lax.axis_index("c")
        s = lax.axis_index("s")
        pltpu.sync_copy(zeros_hbm, acc_sh.at[pl.ds(s * RPS, RPS)])

        def run(nch, base):
            pltpu.sync_copy(srcr_hbm.at[pl.ds(base, nch)],
                            src_v.at[pl.ds(0, nch)])
            pltpu.sync_copy(dstr_hbm.at[pl.ds(base, nch)],
                            dst_v.at[pl.ds(0, nch)])
            plsc.subcore_barrier()
            for k in range(GDEPTH):
                pltpu.async_copy(hs_hbm.at[src_v.at[k]], rows_v.at[k],
                                 gsems[k])

            @pl.loop(0, nch, step=NBUF)
            def _(j0):
                for b in range(NBUF):
                    j = j0 + b
                    pltpu.make_async_copy(
                        hs_hbm.at[src_v.at[j]], rows_v.at[b], gsems[b]).wait()
                    pltpu.async_copy(
                        rows_v.at[b], acc_sh.at[dst_v.at[j]], ssems[b],
                        add=True)
                    bn = (b + GDEPTH) % NBUF
                    jn = j + GDEPTH

                    @pl.when(jn < nch)
                    def _():
                        @pl.when(jn >= NBUF)
                        def _():
                            pltpu.make_async_copy(
                                rows_v.at[bn], acc_sh.at[dst_v.at[j]],
                                ssems[bn]).wait()

                        pltpu.async_copy(
                            hs_hbm.at[src_v.at[jn]], rows_v.at[bn], gsems[bn])

            for b in range(NBUF):
                pltpu.make_async_copy(
                    rows_v.at[b], acc_sh.at[dst_v.at[b]], ssems[b]).wait()

        @pl.when(c == SLOW)
        def _():
            run(na, s * na)

        @pl.when(c != SLOW)
        def _():
            run(nb, NS * na + s * nb)

        plsc.subcore_barrier()
        pltpu.sync_copy(acc_sh.at[pl.ds(s * RPS, RPS)],
                        out_hbm.at[pl.ds(c * ACC_N + s * RPS, RPS)])

    return agg


_agg = {64: _make_agg(64, 80, 80, 4, 4),
        32: _make_agg(32, 80, 80, 8, 6),
        16: _make_agg(16, 80, 80, 8, 6)}


def _row_spec(width):
    return pl.BlockSpec((ROW_BLK, width), lambda i: (i, 0))


def _full_spec(shape):
    return pl.BlockSpec(shape, lambda i: (0, 0))


def _tc_call(body, in_specs, out_widths):
    out_shape = tuple(jax.ShapeDtypeStruct((N, w), jnp.float32) for w in out_widths)
    out_specs = tuple(_row_spec(w) for w in out_widths)
    if len(out_widths) == 1:
        out_shape, out_specs = out_shape[0], out_specs[0]
    return pl.pallas_call(
        body, grid=(GRID,), in_specs=in_specs,
        out_specs=out_specs, out_shape=out_shape)


def _t1_body(x_ref, w_ref, o_ref):
    o_ref[...] = jnp.dot(x_ref[...], w_ref[...],
                         preferred_element_type=jnp.float32)


def _f1_body(d0_ref, d1_ref, t1_ref, o_dinv, o_f1):
    deg = d0_ref[...] + d1_ref[...] + 1.0
    dinv = lax.rsqrt(deg)
    o_dinv[...] = dinv
    o_f1[...] = dinv * t1_ref[...]


def _post1_body(a0, a1, t1, dinv, b1, o_h1, o_f2):
    dv = dinv[...]
    g = dv * (a0[...] + a1[...]) + dv * dv * t1[...] + b1[...]
    h1 = jnp.maximum(g, 0.0)
    o_h1[...] = h1
    o_f2[...] = dv * h1


def _post2_body(a0, a1, h1, dinv, w2, b2, w3, o_t3, o_f3):
    dv = dinv[...]
    g2 = dv * (a0[...] + a1[...]) + dv * dv * h1[...]
    h2 = jnp.maximum(
        jnp.dot(g2, w2[...], preferred_element_type=jnp.float32) + b2[...], 0.0)
    t3 = jnp.dot(h2, w3[...], preferred_element_type=jnp.float32)
    o_t3[...] = t3
    o_f3[...] = dv * t3


def _post3_body(a0, a1, t3, dinv, b3, w4, o_t4, o_f4):
    dv = dinv[...]
    h3 = jnp.maximum(dv * (a0[...] + a1[...]) + dv * dv * t3[...] + b3[...], 0.0)
    t4 = jnp.dot(h3, w4[...], preferred_element_type=jnp.float32)
    o_t4[...] = t4
    o_f4[...] = dv * t4


def _post4_body(a0, a1, t4, dinv, b4, o_h4, o_f5):
    dv = dinv[...]
    h4 = jnp.maximum(dv * (a0[...] + a1[...]) + dv * dv * t4[...] + b4[...], 0.0)
    o_h4[...] = h4
    o_f5[...] = dv * h4


def _post5_body(a0, a1, h4, dinv, w5, b5, o_ref):
    dv = dinv[...]
    g5 = dv * (a0[...] + a1[...]) + dv * dv * h4[...]
    o_ref[...] = jnp.dot(g5, w5[...], preferred_element_type=jnp.float32) + b5[...]


def kernel(x, edge_index, W1, b1, W2, b2, W3, b3, W4, b4, W5, b5):
    src = edge_index[0]
    dst = edge_index[1]
    pad = EPAD - E
    srcr = jnp.concatenate(
        [src, jnp.zeros((pad,), src.dtype)]).reshape(NW * NCH, CH)
    dstr = jnp.concatenate(
        [dst, jnp.full((pad,), N, dst.dtype)]).reshape(NW * NCH, CH)

    zeros64 = jnp.zeros((RPS, 64), jnp.float32)
    zeros32 = jnp.zeros((RPS, 32), jnp.float32)
    zeros16 = jnp.zeros((RPS, 16), jnp.float32)
    ones16 = jnp.ones((N, 16), jnp.float32)

    degp = _agg[16](ones16, srcr, dstr, zeros16)
    t1 = _tc_call(_t1_body, [_row_spec(128), _full_spec((128, 64))], (64,))(x, W1)

    d0 = degp[:N, :1]
    d1 = degp[ACC_N:ACC_N + N, :1]
    dinv, f1 = _tc_call(
        _f1_body, [_row_spec(1), _row_spec(1), _row_spec(64)], (1, 64),
    )(d0, d1, t1)

    a1 = _agg[64](f1, srcr, dstr, zeros64)
    h1, f2 = _tc_call(
        _post1_body,
        [_row_spec(64), _row_spec(64), _row_spec(64), _row_spec(1),
         _full_spec((1, 64))],
        (64, 64),
    )(a1[:N], a1[ACC_N:ACC_N + N], t1, dinv, b1.reshape(1, -1))

    a2 = _agg[64](f2, srcr, dstr, zeros64)
    t3, f3 = _tc_call(
        _post2_body,
        [_row_spec(64), _row_spec(64), _row_spec(64), _row_spec(1),
         _full_spec((64, 256)), _full_spec((1, 256)), _full_spec((256, 32))],
        (32, 32),
    )(a2[:N], a2[ACC_N:ACC_N + N], h1, dinv, W2, b2.reshape(1, -1), W3)

    a3 = _agg[32](f3, srcr, dstr, zeros32)
    t4, f4 = _tc_call(
        _post3_body,
        [_row_spec(32), _row_spec(32), _row_spec(32), _row_spec(1),
         _full_spec((1, 32)), _full_spec((32, 16))],
        (16, 16),
    )(a3[:N], a3[ACC_N:ACC_N + N], t3, dinv, b3.reshape(1, -1), W4)

    a4 = _agg[16](f4, srcr, dstr, zeros16)
    h4, f5 = _tc_call(
        _post4_body,
        [_row_spec(16), _row_spec(16), _row_spec(16), _row_spec(1),
         _full_spec((1, 16))],
        (16, 16),
    )(a4[:N], a4[ACC_N:ACC_N + N], t4, dinv, b4.reshape(1, -1))

    a5 = _agg[16](f5, srcr, dstr, zeros16)
    out = _tc_call(
        _post5_body,
        [_row_spec(16), _row_spec(16), _row_spec(16), _row_spec(1),
         _full_spec((16, 40)), _full_spec((1, 40))],
        (40,),
    )(a5[:N], a5[ACC_N:ACC_N + N], h4, dinv, W5, b5.reshape(1, -1))
    return out


# speedup vs baseline: 1.1280x; 1.0016x over previous
import functools

import jax
import jax.numpy as jnp
from jax import lax
from jax.experimental import pallas as pl
from jax.experimental.pallas import tpu as pltpu
from jax.experimental.pallas import tpu_sc as plsc

N = 10000
E = 320000
NC, NS = 2, 16
NW = NC * NS
CH = 128
NCH = 80
EPAD = NW * NCH * CH
ACC_N = 10240
RPS = ACC_N // NS
ACC_ROWS = ACC_N

NCHT = 160
SLOW = 0

ROW_BLK = 2000
GRID = N // ROW_BLK

_sc_mesh = plsc.VectorSubcoreMesh(core_axis_name="c", subcore_axis_name="s")
_sc_params = pltpu.CompilerParams(use_tc_tiling_on_sc=False)


def _make_agg(F, na, nb, NBUF, GDEPTH):
    assert na % NBUF == 0 and nb % NBUF == 0 and na + nb == NCHT

    @functools.partial(
        pl.kernel,
        out_type=jax.ShapeDtypeStruct((NC * ACC_N, F), jnp.float32),
        mesh=_sc_mesh,
        compiler_params=_sc_params,
        scratch_types=[
            pltpu.VMEM((max(na, nb), CH), jnp.int32),
            pltpu.VMEM((max(na, nb), CH), jnp.int32),
            pltpu.VMEM((NBUF, CH, F), jnp.float32),
            pltpu.VMEM_SHARED((ACC_ROWS, F), jnp.float32),
        ] + [pltpu.SemaphoreType.DMA] * (2 * NBUF),
    )
    def agg(hs_hbm, srcr_hbm, dstr_hbm, zeros_hbm, out_hbm,
            src_v, dst_v, rows_v, acc_sh, *sems):
        gsems, ssems = sems[:NBUF], sems[NBUF:]
        c = lax.axis_index("c")
        s = lax.axis_index("s")
        pltpu.sync_copy(zeros_hbm, acc_sh.at[pl.ds(s * RPS, RPS)])

        def run(nch, base):
            pltpu.sync_copy(srcr_hbm.at[pl.ds(base, nch)],
                            src_v.at[pl.ds(0, nch)])
            pltpu.sync_copy(dstr_hbm.at[pl.ds(base, nch)],
                            dst_v.at[pl.ds(0, nch)])
            plsc.subcore_barrier()
            for k in range(GDEPTH):
                pltpu.async_copy(hs_hbm.at[src_v.at[k]], rows_v.at[k],
                                 gsems[k])

            @pl.loop(0, nch, step=NBUF)
            def _(j0):
                for b in range(NBUF):
                    j = j0 + b
                    pltpu.make_async_copy(
                        hs_hbm.at[src_v.at[j]], rows_v.at[b], gsems[b]).wait()
                    pltpu.async_copy(
                        rows_v.at[b], acc_sh.at[dst_v.at[j]], ssems[b],
                        add=True)
                    bn = (b + GDEPTH) % NBUF
                    jn = j + GDEPTH

                    @pl.when(jn < nch)
                    def _():
                        @pl.when(jn >= NBUF)
                        def _():
                            pltpu.make_async_copy(
                                rows_v.at[bn], acc_sh.at[dst_v.at[j]],
                                ssems[bn]).wait()

                        pltpu.async_copy(
                            hs_hbm.at[src_v.at[jn]], rows_v.at[bn], gsems[bn])

            for b in range(NBUF):
                pltpu.make_async_copy(
                    rows_v.at[b], acc_sh.at[dst_v.at[b]], ssems[b]).wait()

        @pl.when(c == SLOW)
        def _():
            run(na, s * na)

        @pl.when(c != SLOW)
        def _():
            run(nb, NS * na + s * nb)

        plsc.subcore_barrier()
        pltpu.sync_copy(acc_sh.at[pl.ds(s * RPS, RPS)],
                        out_hbm.at[pl.ds(c * ACC_N + s * RPS, RPS)])

    return agg


_agg = {64: _make_agg(64, 80, 80, 4, 4),
        32: _make_agg(32, 80, 80, 4, 4),
        16: _make_agg(16, 80, 80, 4, 4)}


def _row_spec(width):
    return pl.BlockSpec((ROW_BLK, width), lambda i: (i, 0))


def _full_spec(shape):
    return pl.BlockSpec(shape, lambda i: (0, 0))


def _tc_call(body, in_specs, out_widths):
    out_shape = tuple(jax.ShapeDtypeStruct((N, w), jnp.float32) for w in out_widths)
    out_specs = tuple(_row_spec(w) for w in out_widths)
    if len(out_widths) == 1:
        out_shape, out_specs = out_shape[0], out_specs[0]
    return pl.pallas_call(
        body, grid=(GRID,), in_specs=in_specs,
        out_specs=out_specs, out_shape=out_shape)


def _t1_body(x_ref, w_ref, o_ref):
    o_ref[...] = jnp.dot(x_ref[...], w_ref[...],
                         preferred_element_type=jnp.float32)


def _f1_body(d0_ref, d1_ref, t1_ref, o_dinv, o_f1):
    deg = d0_ref[...] + d1_ref[...] + 1.0
    dinv = lax.rsqrt(deg)
    o_dinv[...] = dinv
    o_f1[...] = dinv * t1_ref[...]


def _post1_body(a0, a1, t1, dinv, b1, o_h1, o_f2):
    dv = dinv[...]
    g = dv * (a0[...] + a1[...]) + dv * dv * t1[...] + b1[...]
    h1 = jnp.maximum(g, 0.0)
    o_h1[...] = h1
    o_f2[...] = dv * h1


def _post2_body(a0, a1, h1, dinv, w2, b2, w3, o_t3, o_f3):
    dv = dinv[...]
    g2 = dv * (a0[...] + a1[...]) + dv * dv * h1[...]
    h2 = jnp.maximum(
        jnp.dot(g2, w2[...], preferred_element_type=jnp.float32) + b2[...], 0.0)
    t3 = jnp.dot(h2, w3[...], preferred_element_type=jnp.float32)
    o_t3[...] = t3
    o_f3[...] = dv * t3


def _post3_body(a0, a1, t3, dinv, b3, w4, o_t4, o_f4):
    dv = dinv[...]
    h3 = jnp.maximum(dv * (a0[...] + a1[...]) + dv * dv * t3[...] + b3[...], 0.0)
    t4 = jnp.dot(h3, w4[...], preferred_element_type=jnp.float32)
    o_t4[...] = t4
    o_f4[...] = dv * t4


def _post4_body(a0, a1, t4, dinv, b4, o_h4, o_f5):
    dv = dinv[...]
    h4 = jnp.maximum(dv * (a0[...] + a1[...]) + dv * dv * t4[...] + b4[...], 0.0)
    o_h4[...] = h4
    o_f5[...] = dv * h4


def _post5_body(a0, a1, h4, dinv, w5, b5, o_ref):
    dv = dinv[...]
    g5 = dv * (a0[...] + a1[...]) + dv * dv * h4[...]
    o_ref[...] = jnp.dot(g5, w5[...], preferred_element_type=jnp.float32) + b5[...]


def kernel(x, edge_index, W1, b1, W2, b2, W3, b3, W4, b4, W5, b5):
    src = edge_index[0]
    dst = edge_index[1]
    pad = EPAD - E
    srcr = jnp.concatenate(
        [src, jnp.zeros((pad,), src.dtype)]).reshape(NW * NCH, CH)
    dstr = jnp.concatenate(
        [dst, jnp.full((pad,), N, dst.dtype)]).reshape(NW * NCH, CH)

    zeros64 = jnp.zeros((RPS, 64), jnp.float32)
    zeros32 = jnp.zeros((RPS, 32), jnp.float32)
    zeros16 = jnp.zeros((RPS, 16), jnp.float32)
    ones16 = jnp.ones((N, 16), jnp.float32)

    degp = _agg[16](ones16, srcr, dstr, zeros16)
    t1 = _tc_call(_t1_body, [_row_spec(128), _full_spec((128, 64))], (64,))(x, W1)

    d0 = degp[:N, :1]
    d1 = degp[ACC_N:ACC_N + N, :1]
    dinv, f1 = _tc_call(
        _f1_body, [_row_spec(1), _row_spec(1), _row_spec(64)], (1, 64),
    )(d0, d1, t1)

    a1 = _agg[64](f1, srcr, dstr, zeros64)
    h1, f2 = _tc_call(
        _post1_body,
        [_row_spec(64), _row_spec(64), _row_spec(64), _row_spec(1),
         _full_spec((1, 64))],
        (64, 64),
    )(a1[:N], a1[ACC_N:ACC_N + N], t1, dinv, b1.reshape(1, -1))

    a2 = _agg[64](f2, srcr, dstr, zeros64)
    t3, f3 = _tc_call(
        _post2_body,
        [_row_spec(64), _row_spec(64), _row_spec(64), _row_spec(1),
         _full_spec((64, 256)), _full_spec((1, 256)), _full_spec((256, 32))],
        (32, 32),
    )(a2[:N], a2[ACC_N:ACC_N + N], h1, dinv, W2, b2.reshape(1, -1), W3)

    a3 = _agg[32](f3, srcr, dstr, zeros32)
    t4, f4 = _tc_call(
        _post3_body,
        [_row_spec(32), _row_spec(32), _row_spec(32), _row_spec(1),
         _full_spec((1, 32)), _full_spec((32, 16))],
        (16, 16),
    )(a3[:N], a3[ACC_N:ACC_N + N], t3, dinv, b3.reshape(1, -1), W4)

    a4 = _agg[16](f4, srcr, dstr, zeros16)
    h4, f5 = _tc_call(
        _post4_body,
        [_row_spec(16), _row_spec(16), _row_spec(16), _row_spec(1),
         _full_spec((1, 16))],
        (16, 16),
    )(a4[:N], a4[ACC_N:ACC_N + N], t4, dinv, b4.reshape(1, -1))

    a5 = _agg[16](f5, srcr, dstr, zeros16)
    out = _tc_call(
        _post5_body,
        [_row_spec(16), _row_spec(16), _row_spec(16), _row_spec(1),
         _full_spec((16, 40)), _full_spec((1, 40))],
        (40,),
    )(a5[:N], a5[ACC_N:ACC_N + N], h4, dinv, W5, b5.reshape(1, -1))
    return out
